# baseline (speedup 1.0000x reference)
import functools

import jax
import jax.numpy as jnp
from jax import lax
from jax.experimental import pallas as pl
from jax.experimental.pallas import tpu as pltpu
from jax.experimental.pallas import tpu_sc as plsc

_D = 256
_K = 1024
_L = 96
_B = 64
_B_BLK = 32
_N = _B_BLK * _L
_SCALE = 1.25 / (_B * _L * _D)

_NW = 32
_ROWS = _B * _L
_BPW = _ROWS // _NW
_CHUNK = 96
_NCHUNK = _BPW // _CHUNK


def _tc_body(x_ref, cb_ref, idx_ref, loss_ref, c2_ref):
    i = pl.program_id(0)
    cb = cb_ref[...]

    @pl.when(i == 0)
    def _prep():
        c2 = jnp.sum(cb * cb, axis=1, keepdims=True)
        c2_ref[...] = 0.5 * jnp.broadcast_to(c2, (_K, 128))

    xcat = jnp.concatenate([x_ref[b][:, :_L] for b in range(_B_BLK)], axis=1)
    ip = jnp.dot(cb, xcat, preferred_element_type=jnp.float32)
    dist = c2_ref[:, :1] - ip
    idx = jnp.argmin(dist, axis=0)
    idx_ref[0, 0] = idx

    x2 = jnp.sum(xcat * xcat, axis=0, keepdims=True)
    mind = jnp.min(dist, axis=0, keepdims=True)
    part = jnp.sum(x2 + 2.0 * mind)

    @pl.when(i == 0)
    def _init():
        loss_ref[0, 0] = part

    @pl.when(i > 0)
    def _acc():
        loss_ref[0, 0] += part

    @pl.when(i == (_B // _B_BLK) - 1)
    def _final():
        loss_ref[0, 0] *= _SCALE


_sc_mesh = plsc.VectorSubcoreMesh(core_axis_name="c", subcore_axis_name="s")


@functools.partial(
    pl.kernel,
    mesh=_sc_mesh,
    out_type=jax.ShapeDtypeStruct((_ROWS, _D), jnp.float32),
    scratch_types=[
        pltpu.VMEM((_CHUNK,), jnp.int32),
        pltpu.VMEM((_CHUNK, _D), jnp.float32),
        pltpu.SemaphoreType.DMA,
    ],
)
def _sc_gather(cb_hbm, idx_hbm, out_hbm, idx_v, rows_v, sem):
    wid = lax.axis_index("s") * 2 + lax.axis_index("c")
    base = wid * _BPW
    for c in range(_NCHUNK):
        off = base + c * _CHUNK
        pltpu.sync_copy(idx_hbm.at[pl.ds(off, _CHUNK)], idx_v)
        pltpu.async_copy(cb_hbm.at[idx_v], rows_v, sem).wait()
        pltpu.sync_copy(rows_v, out_hbm.at[pl.ds(off, _CHUNK)])


def kernel(x, codebook):
    idx3, loss = pl.pallas_call(
        _tc_body,
        grid=(_B // _B_BLK,),
        in_specs=[
            pl.BlockSpec((_B_BLK, _D, 128), lambda i: (i, 0, 0)),
            pl.BlockSpec((_K, _D), lambda i: (0, 0)),
        ],
        out_specs=[
            pl.BlockSpec((1, 1, _N), lambda i: (i, 0, 0)),
            pl.BlockSpec(memory_space=pltpu.SMEM),
        ],
        out_shape=[
            jax.ShapeDtypeStruct((_B // _B_BLK, 1, _N), jnp.int32),
            jax.ShapeDtypeStruct((1, 1), jnp.float32),
        ],
        scratch_shapes=[pltpu.VMEM((_K, 128), jnp.float32)],
    )(x, codebook)
    idx_flat = idx3.reshape(_ROWS)
    q = _sc_gather(codebook, idx_flat)
    q = q.reshape(_B, _L, _D)
    return jnp.transpose(q, (0, 2, 1)), loss[0, 0]


# speedup vs baseline: 1.0159x; 1.0159x over previous
import functools

import jax
import jax.numpy as jnp
from jax import lax
from jax.experimental import pallas as pl
from jax.experimental.pallas import tpu as pltpu
from jax.experimental.pallas import tpu_sc as plsc

_D = 256
_K = 1024
_L = 96
_B = 64
_B_BLK = 32
_N = _B_BLK * _L
_SCALE = 1.25 / (_B * _L * _D)

_NW = 32
_ROWS = _B * _L
_BPW = _ROWS // _NW
_CHUNK = 96
_NCHUNK = _BPW // _CHUNK


def _tc_body(x_ref, cb_ref, idx_ref, loss_ref, c2_ref):
    i = pl.program_id(0)
    cb = cb_ref[...]

    @pl.when(i == 0)
    def _prep():
        c2 = jnp.sum(cb * cb, axis=1, keepdims=True)
        c2_ref[...] = 0.5 * jnp.broadcast_to(c2, (_K, 128))

    xcat = jnp.concatenate([x_ref[b][:, :_L] for b in range(_B_BLK)], axis=1)
    ip = jnp.dot(cb, xcat, preferred_element_type=jnp.float32)
    dist = c2_ref[:, :1] - ip
    idx = jnp.argmin(dist, axis=0)
    idx_ref[0, 0] = idx

    x2 = jnp.sum(xcat * xcat, axis=0, keepdims=True)
    mind = jnp.min(dist, axis=0, keepdims=True)
    part = jnp.sum(x2 + 2.0 * mind)

    @pl.when(i == 0)
    def _init():
        loss_ref[0, 0] = part

    @pl.when(i > 0)
    def _acc():
        loss_ref[0, 0] += part

    @pl.when(i == (_B // _B_BLK) - 1)
    def _final():
        loss_ref[0, 0] *= _SCALE


_sc_mesh = plsc.VectorSubcoreMesh(core_axis_name="c", subcore_axis_name="s")


@functools.partial(
    pl.kernel,
    mesh=_sc_mesh,
    out_type=jax.ShapeDtypeStruct((_ROWS, _D), jnp.float32),
    scratch_types=[
        pltpu.VMEM((_BPW,), jnp.int32),
        pltpu.VMEM((_CHUNK, _D), jnp.float32),
        pltpu.VMEM((_CHUNK, _D), jnp.float32),
        pltpu.SemaphoreType.DMA,
        pltpu.SemaphoreType.DMA,
        pltpu.SemaphoreType.DMA,
        pltpu.SemaphoreType.DMA,
    ],
)
def _sc_gather(cb_hbm, idx_hbm, out_hbm, idx_v, r0, r1, sg0, sg1, sw0, sw1):
    wid = lax.axis_index("s") * 2 + lax.axis_index("c")
    base = wid * _BPW
    pltpu.sync_copy(idx_hbm.at[pl.ds(base, _BPW)], idx_v)
    g0 = pltpu.async_copy(cb_hbm.at[idx_v.at[pl.ds(0, _CHUNK)]], r0, sg0)
    g1 = pltpu.async_copy(cb_hbm.at[idx_v.at[pl.ds(_CHUNK, _CHUNK)]], r1, sg1)
    g0.wait()
    w0 = pltpu.async_copy(r0, out_hbm.at[pl.ds(base, _CHUNK)], sw0)
    g1.wait()
    w1 = pltpu.async_copy(r1, out_hbm.at[pl.ds(base + _CHUNK, _CHUNK)], sw1)
    w0.wait()
    w1.wait()


def kernel(x, codebook):
    idx3, loss = pl.pallas_call(
        _tc_body,
        grid=(_B // _B_BLK,),
        in_specs=[
            pl.BlockSpec((_B_BLK, _D, 128), lambda i: (i, 0, 0)),
            pl.BlockSpec((_K, _D), lambda i: (0, 0)),
        ],
        out_specs=[
            pl.BlockSpec((1, 1, _N), lambda i: (i, 0, 0)),
            pl.BlockSpec(memory_space=pltpu.SMEM),
        ],
        out_shape=[
            jax.ShapeDtypeStruct((_B // _B_BLK, 1, _N), jnp.int32),
            jax.ShapeDtypeStruct((1, 1), jnp.float32),
        ],
        scratch_shapes=[pltpu.VMEM((_K, 128), jnp.float32)],
    )(x, codebook)
    idx_flat = idx3.reshape(_ROWS)
    q = _sc_gather(codebook, idx_flat)
    q = q.reshape(_B, _L, _D)
    return jnp.transpose(q, (0, 2, 1)), loss[0, 0]


# speedup vs baseline: 1.0924x; 1.0753x over previous
import functools

import jax
import jax.numpy as jnp
from jax import lax
from jax.experimental import pallas as pl
from jax.experimental.pallas import tpu as pltpu
from jax.experimental.pallas import tpu_sc as plsc

_D = 256
_K = 1024
_L = 96
_B = 64
_B_SC = 16
_B_BLK = 16
_N = _B_BLK * _L
_SCALE = 1.25 / (_B * _L * _D)

_NW = 32
_ROWS_SC = _B_SC * _L
_BPW = _ROWS_SC // _NW


def _dist_argmin(x_ref, cb, c2_ref):
    xcat = jnp.concatenate([x_ref[b][:, :_L] for b in range(_B_BLK)], axis=1)
    ip = jnp.dot(cb, xcat, preferred_element_type=jnp.float32)
    dist = c2_ref[:, :1] - ip
    idx = jnp.argmin(dist, axis=0)
    x2 = jnp.sum(xcat * xcat, axis=0, keepdims=True)
    mind = jnp.min(dist, axis=0, keepdims=True)
    part = jnp.sum(x2 + 2.0 * mind)
    return dist, idx, part


def _prep_c2(i, cb, c2_ref):
    @pl.when(i == 0)
    def _c2():
        c2 = jnp.sum(cb * cb, axis=1, keepdims=True)
        c2_ref[...] = 0.5 * jnp.broadcast_to(c2, (_K, 128))


def _acc_loss(i, last, part, loss_ref):
    @pl.when(i == 0)
    def _init():
        loss_ref[0, 0] = part

    @pl.when(i > 0)
    def _acc():
        loss_ref[0, 0] += part


def _tc_idx_body(x_ref, cb_ref, idx_ref, loss_ref, c2_ref):
    i = pl.program_id(0)
    cb = cb_ref[...]
    _prep_c2(i, cb, c2_ref)
    _, idx, part = _dist_argmin(x_ref, cb, c2_ref)
    idx_ref[0, 0] = idx
    _acc_loss(i, _B_SC // _B_BLK - 1, part, loss_ref)


def _tc_full_body(x_ref, cb_ref, q_ref, loss_ref, c2_ref, cb16_ref):
    i = pl.program_id(0)
    cb = cb_ref[...]
    _prep_c2(i, cb, c2_ref)

    @pl.when(i == 0)
    def _cb16():
        cb16_ref[...] = cb.astype(jnp.bfloat16)

    _, idx, part = _dist_argmin(x_ref, cb, c2_ref)
    onehot = (jax.lax.broadcasted_iota(jnp.int32, (_K, _N), 0)
              == idx[None, :]).astype(jnp.bfloat16)
    q = jax.lax.dot_general(onehot, cb16_ref[...], (((0,), (0,)), ((), ())),
                            preferred_element_type=jnp.float32)
    for b in range(_B_BLK):
        q_ref[b] = q[b * _L:(b + 1) * _L, :]
    _acc_loss(i, (_B - _B_SC) // _B_BLK - 1, part, loss_ref)


_sc_mesh = plsc.VectorSubcoreMesh(core_axis_name="c", subcore_axis_name="s")


@functools.partial(
    pl.kernel,
    mesh=_sc_mesh,
    out_type=jax.ShapeDtypeStruct((_ROWS_SC, _D), jnp.float32),
    scratch_types=[
        pltpu.VMEM((_BPW,), jnp.int32),
        pltpu.VMEM((_BPW, _D), jnp.float32),
        pltpu.SemaphoreType.DMA,
        pltpu.SemaphoreType.DMA,
    ],
)
def _sc_gather(cb_hbm, idx_hbm, out_hbm, idx_v, rows_v, sg, sw):
    wid = lax.axis_index("s") * 2 + lax.axis_index("c")
    base = wid * _BPW
    pltpu.sync_copy(idx_hbm.at[pl.ds(base, _BPW)], idx_v)
    pltpu.async_copy(cb_hbm.at[idx_v], rows_v, sg).wait()
    pltpu.async_copy(rows_v, out_hbm.at[pl.ds(base, _BPW)], sw).wait()


def kernel(x, codebook):
    idx3, loss_a = pl.pallas_call(
        _tc_idx_body,
        grid=(_B_SC // _B_BLK,),
        in_specs=[
            pl.BlockSpec((_B_BLK, _D, 128), lambda i: (i, 0, 0)),
            pl.BlockSpec((_K, _D), lambda i: (0, 0)),
        ],
        out_specs=[
            pl.BlockSpec((1, 1, _N), lambda i: (i, 0, 0)),
            pl.BlockSpec(memory_space=pltpu.SMEM),
        ],
        out_shape=[
            jax.ShapeDtypeStruct((_B_SC // _B_BLK, 1, _N), jnp.int32),
            jax.ShapeDtypeStruct((1, 1), jnp.float32),
        ],
        scratch_shapes=[pltpu.VMEM((_K, 128), jnp.float32)],
    )(x, codebook)
    qa = _sc_gather(codebook, idx3.reshape(_ROWS_SC))
    qb, loss_b = pl.pallas_call(
        _tc_full_body,
        grid=((_B - _B_SC) // _B_BLK,),
        in_specs=[
            pl.BlockSpec((_B_BLK, _D, 128),
                         lambda i: (i + _B_SC // _B_BLK, 0, 0)),
            pl.BlockSpec((_K, _D), lambda i: (0, 0)),
        ],
        out_specs=[
            pl.BlockSpec((_B_BLK, _L, _D), lambda i: (i, 0, 0)),
            pl.BlockSpec(memory_space=pltpu.SMEM),
        ],
        out_shape=[
            jax.ShapeDtypeStruct((_B - _B_SC, _L, _D), jnp.float32),
            jax.ShapeDtypeStruct((1, 1), jnp.float32),
        ],
        scratch_shapes=[pltpu.VMEM((_K, 128), jnp.float32),
                        pltpu.VMEM((_K, _D), jnp.bfloat16)],
    )(x, codebook)
    q = jnp.concatenate([qa.reshape(_B_SC, _L, _D), qb], axis=0)
    loss = (loss_a[0, 0] + loss_b[0, 0]) * _SCALE
    return jnp.transpose(q, (0, 2, 1)), loss


# speedup vs baseline: 2.7918x; 2.5556x over previous
import jax
import jax.numpy as jnp
from jax.experimental import pallas as pl
from jax.experimental.pallas import tpu as pltpu

_D = 256
_K = 1024
_L = 96
_B = 64
_B_BLK = 32
_N = _B_BLK * _L
_SCALE = 1.25 / (_B * _L * _D)


def _vq_body(x_ref, cb_ref, q_ref, loss_ref, c2_ref, cb16_ref):
    i = pl.program_id(0)
    cb = cb_ref[...]

    @pl.when(i == 0)
    def _prep():
        c2 = jnp.sum(cb * cb, axis=1, keepdims=True)
        c2_ref[...] = 0.5 * jnp.broadcast_to(c2, (_K, 128))
        cb16_ref[...] = cb.astype(jnp.bfloat16)

    xcat = jnp.concatenate([x_ref[b][:, :_L] for b in range(_B_BLK)], axis=1)
    ip = jnp.dot(cb, xcat, preferred_element_type=jnp.float32)
    dist = c2_ref[:, :1] - ip
    idx = jnp.argmin(dist, axis=0)
    onehot = (jax.lax.broadcasted_iota(jnp.int32, (_K, _N), 0)
              == idx[None, :]).astype(jnp.bfloat16)
    q = jax.lax.dot_general(onehot, cb16_ref[...], (((0,), (0,)), ((), ())),
                            preferred_element_type=jnp.float32)
    for b in range(_B_BLK):
        q_ref[b] = q[b * _L:(b + 1) * _L, :]

    x2 = jnp.sum(xcat * xcat, axis=0, keepdims=True)
    mind = jnp.min(dist, axis=0, keepdims=True)
    part = jnp.sum(x2 + 2.0 * mind)

    @pl.when(i == 0)
    def _init():
        loss_ref[0, 0] = part

    @pl.when(i > 0)
    def _acc():
        loss_ref[0, 0] += part

    @pl.when(i == (_B // _B_BLK) - 1)
    def _final():
        loss_ref[0, 0] *= _SCALE


def kernel(x, codebook):
    q, loss = pl.pallas_call(
        _vq_body,
        grid=(_B // _B_BLK,),
        in_specs=[
            pl.BlockSpec((_B_BLK, _D, 128), lambda i: (i, 0, 0)),
            pl.BlockSpec((_K, _D), lambda i: (0, 0)),
        ],
        out_specs=[
            pl.BlockSpec((_B_BLK, _L, _D), lambda i: (i, 0, 0)),
            pl.BlockSpec(memory_space=pltpu.SMEM),
        ],
        out_shape=[
            jax.ShapeDtypeStruct((_B, _L, _D), jnp.float32),
            jax.ShapeDtypeStruct((1, 1), jnp.float32),
        ],
        scratch_shapes=[pltpu.VMEM((_K, 128), jnp.float32),
                        pltpu.VMEM((_K, _D), jnp.bfloat16)],
    )(x, codebook)
    return jnp.transpose(q, (0, 2, 1)), loss[0, 0]
